# baseline (speedup 1.0000x reference)
import functools

import jax
import jax.numpy as jnp
from jax import lax
from jax.experimental import pallas as pl
from jax.experimental.pallas import tpu as pltpu
from jax.experimental.pallas import tpu_sc as plsc

_N = 10000
_E = 320000
_D = 128
_H = 32
_O = 16

_NC = 2
_NS = 16
_NW = _NC * _NS
_CH = 128
_NCHUNK = 81
_EPT = _NCHUNK * _CH
_EPAD = _NW * _EPT
_NP = 10240
_RPT = _NP // _NS
_RPTP = _NP // _NS

_mesh = plsc.VectorSubcoreMesh(core_axis_name="c", subcore_axis_name="s")
_sc_params = pltpu.CompilerParams(use_tc_tiling_on_sc=False,
                                  skip_device_barrier=True)


@functools.partial(
    pl.kernel,
    out_type=jax.ShapeDtypeStruct((_NC * 3 * _NP,), jnp.float32),
    mesh=_mesh,
    compiler_params=_sc_params,
    scratch_types=[
        pltpu.VMEM((_NCHUNK, _CH), jnp.int32),
        pltpu.VMEM((_NCHUNK, _CH), jnp.float32),
        pltpu.VMEM_SHARED((_NP,), jnp.float32),
        pltpu.VMEM_SHARED((_NP,), jnp.float32),
        pltpu.VMEM_SHARED((_NP,), jnp.float32),
        pltpu.SemaphoreType.DMA,
        pltpu.SemaphoreType.DMA,
        pltpu.SemaphoreType.DMA,
        pltpu.SemaphoreType.DMA,
    ],
)
def _sc_degrees(c1, c2, c3, e1, e2, e3, zeros_np, deg_out, col_v, ew_v,
                acc1, acc2, acc3, d0, d1, d2, d3):
    cid = lax.axis_index("c")
    sid = lax.axis_index("s")
    wid = sid * _NC + cid
    accs = (acc1, acc2, acc3)
    for k in range(3):
        pltpu.sync_copy(zeros_np.at[pl.ds(sid * _RPTP, _RPTP)],
                        accs[k].at[pl.ds(sid * _RPTP, _RPTP)])
    plsc.subcore_barrier()
    dsems = (d0, d1, d2, d3)
    for k, (col_h, ew_h) in enumerate(((c1, e1), (c2, e2), (c3, e3))):
        pltpu.sync_copy(col_h.at[wid], col_v)
        pltpu.sync_copy(ew_h.at[wid], ew_v)
        acc_k = accs[k]

        def dwait(sem, acc_k=acc_k):
            pltpu.make_async_copy(ew_v.at[0], acc_k.at[col_v.at[0]],
                                  sem).wait()

        @pl.loop(0, (_NCHUNK - 1) // 4)
        def _(p, acc_k=acc_k):
            for b in range(4):
                j = 4 * p + b

                @pl.when(p >= 1)
                def _(b=b):
                    dwait(dsems[b])

                pltpu.async_copy(ew_v.at[j], acc_k.at[col_v.at[j]],
                                 dsems[b], add=True)

        jt = _NCHUNK - 1
        dwait(d0)
        pltpu.async_copy(ew_v.at[jt], acc_k.at[col_v.at[jt]], d0, add=True)
        for sem in (d0, d1, d2, d3):
            dwait(sem)

    plsc.subcore_barrier()
    for k in range(3):
        pltpu.sync_copy(accs[k].at[pl.ds(sid * _RPTP, _RPTP)],
                        deg_out.at[pl.ds((cid * 3 + k) * _NP + sid * _RPTP,
                                         _RPTP)])


def _make_sc_msg(width):
    @functools.partial(
        pl.kernel,
        out_type=jax.ShapeDtypeStruct((_NC * 3, _NP, width), jnp.float32),
        mesh=_mesh,
        compiler_params=_sc_params,
        scratch_types=[
            pltpu.VMEM((_NCHUNK, _CH), jnp.int32),
            pltpu.VMEM((_NCHUNK, _CH), jnp.int32),
            pltpu.VMEM((_NCHUNK, _CH), jnp.float32),
            pltpu.VMEM((_CH, width), jnp.float32),
            pltpu.VMEM((_CH, width), jnp.float32),
            pltpu.VMEM((_CH, width), jnp.float32),
            pltpu.VMEM((_CH, width), jnp.float32),
            pltpu.VMEM_SHARED((_NP, width), jnp.float32),
            pltpu.VMEM_SHARED((_NP, width), jnp.float32),
            pltpu.VMEM_SHARED((_NP, width), jnp.float32),
            pltpu.SemaphoreType.DMA,
            pltpu.SemaphoreType.DMA,
            pltpu.SemaphoreType.DMA,
            pltpu.SemaphoreType.DMA,
            pltpu.SemaphoreType.DMA,
            pltpu.SemaphoreType.DMA,
            pltpu.SemaphoreType.DMA,
            pltpu.SemaphoreType.DMA,
        ],
    )
    def sc_msg(t1, t2, t3, r1, r2, r3, c1, c2, c3, e1, e2, e3, zeros_w,
               out, row_v, col_v, ew_v, b0, b1, b2, b3, acc1, acc2, acc3,
               g0, g1, g2, g3, s0, s1, s2, s3):
        cid = lax.axis_index("c")
        sid = lax.axis_index("s")
        wid = sid * _NC + cid
        accs = (acc1, acc2, acc3)
        for k in range(3):
            pltpu.sync_copy(zeros_w.at[pl.ds(sid * _RPT, _RPT)],
                            accs[k].at[pl.ds(sid * _RPT, _RPT)])
        plsc.subcore_barrier()
        for k, (tab_h, row_h, col_h, ew_h) in enumerate((
                (t1, r1, c1, e1), (t2, r2, c2, e2), (t3, r3, c3, e3))):
            pltpu.sync_copy(row_h.at[wid], row_v)
            pltpu.sync_copy(col_h.at[wid], col_v)
            pltpu.sync_copy(ew_h.at[wid], ew_v)

            acc_k = accs[k]
            bufs = (b0, b1, b2, b3)
            gsems = (g0, g1, g2, g3)
            ssems = (s0, s1, s2, s3)

            def scale(buf, j):
                @pl.loop(0, _CH, step=16)
                def _(r0):
                    ew16 = ew_v[j, pl.ds(r0, 16)]
                    for i in range(16):
                        s = ew16[i]
                        for f in range(0, width, 16):
                            buf[r0 + i, pl.ds(f, 16)] = (
                                buf[r0 + i, pl.ds(f, 16)] * s)

            def wait_gather(buf, sem, tab_h=tab_h):
                pltpu.make_async_copy(tab_h.at[row_v.at[0]], buf, sem).wait()

            def wait_scatter(buf, sem, acc_k=acc_k):
                pltpu.make_async_copy(buf, acc_k.at[col_v.at[0]], sem).wait()

            pltpu.async_copy(tab_h.at[row_v.at[0]], b0, g0)
            pltpu.async_copy(tab_h.at[row_v.at[1]], b1, g1)

            @pl.loop(0, (_NCHUNK - 1) // 4)
            def _(p, tab_h=tab_h, acc_k=acc_k):
                for b in range(4):
                    j = 4 * p + b
                    buf, gs, ss = bufs[b], gsems[b], ssems[b]
                    nb = (b + 2) % 4
                    wait_gather(buf, gs)
                    scale(buf, j)
                    pltpu.async_copy(buf, acc_k.at[col_v.at[j]], ss, add=True)
                    if b < 2:
                        @pl.when(p >= 1)
                        def _():
                            wait_scatter(bufs[nb], ssems[nb])
                        pltpu.async_copy(tab_h.at[row_v.at[j + 2]],
                                         bufs[nb], gsems[nb])
                    elif b == 2:
                        wait_scatter(bufs[nb], ssems[nb])
                        pltpu.async_copy(tab_h.at[row_v.at[j + 2]],
                                         bufs[nb], gsems[nb])
                    else:
                        wait_scatter(bufs[nb], ssems[nb])

                        @pl.when(p <= (_NCHUNK - 1) // 4 - 2)
                        def _():
                            pltpu.async_copy(tab_h.at[row_v.at[j + 2]],
                                             bufs[nb], gsems[nb])

            jt = _NCHUNK - 1
            wait_gather(b0, g0)
            scale(b0, jt)
            pltpu.async_copy(b0, acc_k.at[col_v.at[jt]], s0, add=True)
            wait_scatter(b0, s0)
            wait_scatter(b2, s2)
            wait_scatter(b3, s3)

        plsc.subcore_barrier()
        for k in range(3):
            pltpu.sync_copy(accs[k].at[pl.ds(sid * _RPT, _RPT)],
                            out.at[cid * 3 + k, pl.ds(sid * _RPT, _RPT)])

    return sc_msg


_sc_msg32 = _make_sc_msg(_H)
_sc_msg16 = _make_sc_msg(_O)


_GRID = 20
_BR = 512


def _tc_prep_body(degp, x1, x2, x3, W1, W2, W3, dinv, xwd1, xwd2, xwd3):
    deg = degp[0] + degp[1] + 1.0
    dv = jnp.where(deg > 0, lax.rsqrt(jnp.maximum(deg, 1e-12)), 0.0)
    dinv[...] = dv
    for k, (x, W, xwd) in enumerate(((x1, W1, xwd1), (x2, W2, xwd2),
                                     (x3, W3, xwd3))):
        xw = jnp.dot(x[...], W[...], preferred_element_type=jnp.float32)
        xwd[...] = xw * dv[k][:, None]


def _tc_prep(degp, x1, x2, x3, W1, W2, W3):
    f32 = jnp.float32
    return pl.pallas_call(
        _tc_prep_body,
        grid=(_GRID,),
        in_specs=[
            pl.BlockSpec((_NC, 3, _BR), lambda i: (0, 0, i)),
            pl.BlockSpec((_BR, _D), lambda i: (i, 0)),
            pl.BlockSpec((_BR, _D), lambda i: (i, 0)),
            pl.BlockSpec((_BR, _D), lambda i: (i, 0)),
            pl.BlockSpec((_D, _H), lambda i: (0, 0)),
            pl.BlockSpec((_D, _H), lambda i: (0, 0)),
            pl.BlockSpec((_D, _H), lambda i: (0, 0)),
        ],
        out_specs=[
            pl.BlockSpec((3, _BR), lambda i: (0, i)),
            pl.BlockSpec((_BR, _H), lambda i: (i, 0)),
            pl.BlockSpec((_BR, _H), lambda i: (i, 0)),
            pl.BlockSpec((_BR, _H), lambda i: (i, 0)),
        ],
        out_shape=[
            jax.ShapeDtypeStruct((3, _NP), f32),
            jax.ShapeDtypeStruct((_N, _H), f32),
            jax.ShapeDtypeStruct((_N, _H), f32),
            jax.ShapeDtypeStruct((_N, _H), f32),
        ],
    )(degp, x1, x2, x3, W1, W2, W3)


def _tc_att_body(accD, xwd1, xwd2, xwd3, dinv, b1, b2, b3, Wf, bf,
                 W11, b11, W22, b22, W33, b33,
                 co1, co2, co3, zd1, zd2, zd3, base):
    embs, coefs, dvs = [], [], []
    for k, (xwd, b) in enumerate(((xwd1, b1), (xwd2, b2), (xwd3, b3))):
        dv = dinv[k][:, None]
        dvs.append(dv)
        pre = (accD[0, k] + accD[1, k] + xwd[...]) * dv + b[...][None, :]
        emb = jnp.maximum(pre, 0.0)
        embs.append(emb)
        s = jnp.sum(emb * Wf[...][:, 0][None, :], axis=1, keepdims=True)
        s = s + bf[...][None, :]
        coefs.append(jnp.exp(jnp.where(s >= 0, s, 0.01 * s)))
    den = coefs[0] + coefs[1] + coefs[2]
    for c_out, c in zip((co1, co2, co3), coefs):
        c_out[...] = c / den
    comb = (coefs[0] * embs[0] + coefs[1] * embs[1] + coefs[2] * embs[2]) / den
    acc_base = (b11[...] + b22[...] + b33[...])[None, :]
    for k, (W, zd_out) in enumerate(((W11, zd1), (W22, zd2), (W33, zd3))):
        z = jnp.dot(comb, W[...], preferred_element_type=jnp.float32)
        zd = z * dvs[k]
        zd_out[...] = zd
        acc_base = acc_base + zd * dvs[k]
    base[...] = acc_base


def _tc_att(accD, xwd1, xwd2, xwd3, dinv, b1, b2, b3, Wf, bf,
            W11, b11, W22, b22, W33, b33):
    f32 = jnp.float32
    vec = lambda n: pl.BlockSpec((n,), lambda i: (0,))
    return pl.pallas_call(
        _tc_att_body,
        grid=(_GRID,),
        in_specs=[
            pl.BlockSpec((_NC, 3, _BR, _H), lambda i: (0, 0, i, 0)),
            pl.BlockSpec((_BR, _H), lambda i: (i, 0)),
            pl.BlockSpec((_BR, _H), lambda i: (i, 0)),
            pl.BlockSpec((_BR, _H), lambda i: (i, 0)),
            pl.BlockSpec((3, _BR), lambda i: (0, i)),
            vec(_H), vec(_H), vec(_H),
            pl.BlockSpec((_H, 1), lambda i: (0, 0)),
            vec(1),
            pl.BlockSpec((_H, _O), lambda i: (0, 0)), vec(_O),
            pl.BlockSpec((_H, _O), lambda i: (0, 0)), vec(_O),
            pl.BlockSpec((_H, _O), lambda i: (0, 0)), vec(_O),
        ],
        out_specs=[
            pl.BlockSpec((_BR, 1), lambda i: (i, 0)),
            pl.BlockSpec((_BR, 1), lambda i: (i, 0)),
            pl.BlockSpec((_BR, 1), lambda i: (i, 0)),
            pl.BlockSpec((_BR, _O), lambda i: (i, 0)),
            pl.BlockSpec((_BR, _O), lambda i: (i, 0)),
            pl.BlockSpec((_BR, _O), lambda i: (i, 0)),
            pl.BlockSpec((_BR, _O), lambda i: (i, 0)),
        ],
        out_shape=[
            jax.ShapeDtypeStruct((_N, 1), f32),
            jax.ShapeDtypeStruct((_N, 1), f32),
            jax.ShapeDtypeStruct((_N, 1), f32),
            jax.ShapeDtypeStruct((_N, _O), f32),
            jax.ShapeDtypeStruct((_N, _O), f32),
            jax.ShapeDtypeStruct((_N, _O), f32),
            jax.ShapeDtypeStruct((_N, _O), f32),
        ],
    )(accD, xwd1, xwd2, xwd3, dinv, b1, b2, b3, Wf, bf,
      W11, b11, W22, b22, W33, b33)


def _tc_final_body(accF, dinv, base, out):
    acc = base[...]
    for k in range(3):
        acc = acc + (accF[0, k] + accF[1, k]) * dinv[k][:, None]
    out[...] = acc


def _tc_final(accF, dinv, base):
    return pl.pallas_call(
        _tc_final_body,
        grid=(_GRID,),
        in_specs=[
            pl.BlockSpec((_NC, 3, _BR, _O), lambda i: (0, 0, i, 0)),
            pl.BlockSpec((3, _BR), lambda i: (0, i)),
            pl.BlockSpec((_BR, _O), lambda i: (i, 0)),
        ],
        out_specs=pl.BlockSpec((_BR, _O), lambda i: (i, 0)),
        out_shape=jax.ShapeDtypeStruct((_N, _O), jnp.float32),
    )(accF, dinv, base)


def kernel(x1, edge_index1, edge_attr1, x2, edge_index2, edge_attr2,
           x3, edge_index3, edge_attr3,
           W1, b1, W2, b2, W3, b3, Wf, bf,
           W11, b11, W22, b22, W33, b33):
    f32 = jnp.float32
    pad = _EPAD - _E
    pad_idx = (jnp.arange(pad, dtype=jnp.int32) * 37) % _N
    rs = lambda a: jnp.concatenate([a, pad_idx]).reshape(_NW, _NCHUNK, _CH)
    rw = lambda a: jnp.concatenate(
        [a, jnp.zeros((pad,), a.dtype)]).reshape(_NW, _NCHUNK, _CH)
    r1, c1 = rs(edge_index1[0]), rs(edge_index1[1])
    r2, c2 = rs(edge_index2[0]), rs(edge_index2[1])
    r3, c3 = rs(edge_index3[0]), rs(edge_index3[1])
    e1, e2, e3 = rw(edge_attr1), rw(edge_attr2), rw(edge_attr3)

    zeros_np = jnp.zeros((_NP,), f32)
    zeros_h = jnp.zeros((_NP, _H), f32)
    zeros_o = jnp.zeros((_NP, _O), f32)

    degp = _sc_degrees(c1, c2, c3, e1, e2, e3, zeros_np).reshape(_NC, 3, _NP)
    dinv, xwd1, xwd2, xwd3 = _tc_prep(degp, x1, x2, x3, W1, W2, W3)
    accD = _sc_msg32(xwd1, xwd2, xwd3, r1, r2, r3, c1, c2, c3,
                     e1, e2, e3, zeros_h).reshape(_NC, 3, _NP, _H)
    co1, co2, co3, zd1, zd2, zd3, base = _tc_att(
        accD, xwd1, xwd2, xwd3, dinv, b1, b2, b3, Wf, bf,
        W11, b11, W22, b22, W33, b33)
    accF = _sc_msg16(zd1, zd2, zd3, r1, r2, r3, c1, c2, c3,
                     e1, e2, e3, zeros_o).reshape(_NC, 3, _NP, _O)
    out4 = _tc_final(accF, dinv, base)
    return (out4, co1, co2, co3)


# speedup vs baseline: 1.2609x; 1.2609x over previous
import functools

import jax
import jax.numpy as jnp
from jax import lax
from jax.experimental import pallas as pl
from jax.experimental.pallas import tpu as pltpu
from jax.experimental.pallas import tpu_sc as plsc

_N = 10000
_E = 320000
_D = 128
_H = 32
_O = 16

_NC = 2
_NS = 16
_NW = _NC * _NS
_CH = 128
_NCHUNK = 81
_EPT = _NCHUNK * _CH
_EPAD = _NW * _EPT
_NP = 10240
_RPT = _NP // _NS
_RPTP = _NP // _NS

_mesh = plsc.VectorSubcoreMesh(core_axis_name="c", subcore_axis_name="s")
_sc_params = pltpu.CompilerParams(use_tc_tiling_on_sc=False)


@functools.partial(
    pl.kernel,
    out_type=jax.ShapeDtypeStruct((_NC * 3 * _NP,), jnp.float32),
    mesh=_mesh,
    compiler_params=_sc_params,
    scratch_types=[
        pltpu.VMEM((_NCHUNK, _CH), jnp.int32),
        pltpu.VMEM((_NCHUNK, _CH), jnp.float32),
        pltpu.VMEM_SHARED((_NP,), jnp.float32),
        pltpu.VMEM_SHARED((_NP,), jnp.float32),
        pltpu.VMEM_SHARED((_NP,), jnp.float32),
        pltpu.SemaphoreType.DMA,
        pltpu.SemaphoreType.DMA,
        pltpu.SemaphoreType.DMA,
        pltpu.SemaphoreType.DMA,
    ],
)
def _sc_degrees(c1, c2, c3, e1, e2, e3, zeros_np, deg_out, col_v, ew_v,
                acc1, acc2, acc3, d0, d1, d2, d3):
    cid = lax.axis_index("c")
    sid = lax.axis_index("s")
    wid = sid * _NC + cid
    accs = (acc1, acc2, acc3)
    for k in range(3):
        pltpu.sync_copy(zeros_np.at[pl.ds(sid * _RPTP, _RPTP)],
                        accs[k].at[pl.ds(sid * _RPTP, _RPTP)])
    plsc.subcore_barrier()
    dsems = (d0, d1, d2, d3)
    for k, (col_h, ew_h) in enumerate(((c1, e1), (c2, e2), (c3, e3))):
        pltpu.sync_copy(col_h.at[wid], col_v)
        pltpu.sync_copy(ew_h.at[wid], ew_v)
        acc_k = accs[k]

        def dwait(sem, acc_k=acc_k):
            pltpu.make_async_copy(ew_v.at[0], acc_k.at[col_v.at[0]],
                                  sem).wait()

        @pl.loop(0, (_NCHUNK - 1) // 4)
        def _(p, acc_k=acc_k):
            for b in range(4):
                j = 4 * p + b

                @pl.when(p >= 1)
                def _(b=b):
                    dwait(dsems[b])

                pltpu.async_copy(ew_v.at[j], acc_k.at[col_v.at[j]],
                                 dsems[b], add=True)

        jt = _NCHUNK - 1
        dwait(d0)
        pltpu.async_copy(ew_v.at[jt], acc_k.at[col_v.at[jt]], d0, add=True)
        for sem in (d0, d1, d2, d3):
            dwait(sem)

    plsc.subcore_barrier()
    for k in range(3):
        pltpu.sync_copy(accs[k].at[pl.ds(sid * _RPTP, _RPTP)],
                        deg_out.at[pl.ds((cid * 3 + k) * _NP + sid * _RPTP,
                                         _RPTP)])


def _make_sc_msg(width):
    @functools.partial(
        pl.kernel,
        out_type=jax.ShapeDtypeStruct((_NC * 3, _NP, width), jnp.float32),
        mesh=_mesh,
        compiler_params=_sc_params,
        scratch_types=[
            pltpu.VMEM((_NCHUNK, _CH), jnp.int32),
            pltpu.VMEM((_NCHUNK, _CH), jnp.int32),
            pltpu.VMEM((_NCHUNK, _CH), jnp.float32),
            *([pltpu.VMEM((_CH, width), jnp.float32)] * 8),
            pltpu.VMEM_SHARED((_NP, width), jnp.float32),
            pltpu.VMEM_SHARED((_NP, width), jnp.float32),
            pltpu.VMEM_SHARED((_NP, width), jnp.float32),
            *([pltpu.SemaphoreType.DMA] * 16),
        ],
    )
    def sc_msg(t1, t2, t3, r1, r2, r3, c1, c2, c3, e1, e2, e3, zeros_w,
               out, row_v, col_v, ew_v,
               b0, b1, b2, b3, b4, b5, b6, b7, acc1, acc2, acc3,
               g0, g1, g2, g3, g4, g5, g6, g7,
               s0, s1, s2, s3, s4, s5, s6, s7):
        cid = lax.axis_index("c")
        sid = lax.axis_index("s")
        wid = sid * _NC + cid
        accs = (acc1, acc2, acc3)
        for k in range(3):
            pltpu.sync_copy(zeros_w.at[pl.ds(sid * _RPT, _RPT)],
                            accs[k].at[pl.ds(sid * _RPT, _RPT)])
        plsc.subcore_barrier()
        for k, (tab_h, row_h, col_h, ew_h) in enumerate((
                (t1, r1, c1, e1), (t2, r2, c2, e2), (t3, r3, c3, e3))):
            pltpu.sync_copy(row_h.at[wid], row_v)
            pltpu.sync_copy(col_h.at[wid], col_v)
            pltpu.sync_copy(ew_h.at[wid], ew_v)

            acc_k = accs[k]
            bufs = (b0, b1, b2, b3, b4, b5, b6, b7)
            gsems = (g0, g1, g2, g3, g4, g5, g6, g7)
            ssems = (s0, s1, s2, s3, s4, s5, s6, s7)

            def scale(buf, j):
                @pl.loop(0, _CH, step=16)
                def _(r0):
                    ew16 = ew_v[j, pl.ds(r0, 16)]
                    for i in range(16):
                        s = ew16[i]
                        for f in range(0, width, 16):
                            buf[r0 + i, pl.ds(f, 16)] = (
                                buf[r0 + i, pl.ds(f, 16)] * s)

            def wait_gather(buf, sem, tab_h=tab_h):
                pltpu.make_async_copy(tab_h.at[row_v.at[0]], buf, sem).wait()

            def wait_scatter(buf, sem, acc_k=acc_k):
                pltpu.make_async_copy(buf, acc_k.at[col_v.at[0]], sem).wait()

            NB = 8
            NP_LOOP = (_NCHUNK - 1) // NB
            for jj in range(6):
                pltpu.async_copy(tab_h.at[row_v.at[jj]], bufs[jj], gsems[jj])

            @pl.loop(0, NP_LOOP)
            def _(p, tab_h=tab_h, acc_k=acc_k):
                for b in range(NB):
                    j = NB * p + b
                    buf, gs, ss = bufs[b], gsems[b], ssems[b]
                    nb = (b + 6) % NB
                    wait_gather(buf, gs)
                    scale(buf, j)
                    pltpu.async_copy(buf, acc_k.at[col_v.at[j]], ss, add=True)
                    if b < 2:
                        @pl.when(p >= 1)
                        def _():
                            wait_scatter(bufs[nb], ssems[nb])
                    else:
                        wait_scatter(bufs[nb], ssems[nb])
                    if b <= 2:
                        pltpu.async_copy(tab_h.at[row_v.at[j + 6]],
                                         bufs[nb], gsems[nb])
                    else:
                        @pl.when(p <= NP_LOOP - 2)
                        def _():
                            pltpu.async_copy(tab_h.at[row_v.at[j + 6]],
                                             bufs[nb], gsems[nb])

            jt = _NCHUNK - 1
            wait_gather(b0, g0)
            scale(b0, jt)
            pltpu.async_copy(b0, acc_k.at[col_v.at[jt]], s0, add=True)
            wait_scatter(b6, s6)
            wait_scatter(b7, s7)
            wait_scatter(b0, s0)

        plsc.subcore_barrier()
        for k in range(3):
            pltpu.sync_copy(accs[k].at[pl.ds(sid * _RPT, _RPT)],
                            out.at[cid * 3 + k, pl.ds(sid * _RPT, _RPT)])

    return sc_msg


_sc_msg32 = _make_sc_msg(_H)
_sc_msg16 = _make_sc_msg(_O)


_GRID = 20
_BR = 512


def _tc_prep_body(degp, x1, x2, x3, W1, W2, W3, dinv, xwd1, xwd2, xwd3):
    deg = degp[0] + degp[1] + 1.0
    dv = jnp.where(deg > 0, lax.rsqrt(jnp.maximum(deg, 1e-12)), 0.0)
    dinv[...] = dv
    for k, (x, W, xwd) in enumerate(((x1, W1, xwd1), (x2, W2, xwd2),
                                     (x3, W3, xwd3))):
        xw = jnp.dot(x[...], W[...], preferred_element_type=jnp.float32)
        xwd[...] = xw * dv[k][:, None]


def _tc_prep(degp, x1, x2, x3, W1, W2, W3):
    f32 = jnp.float32
    return pl.pallas_call(
        _tc_prep_body,
        grid=(_GRID,),
        in_specs=[
            pl.BlockSpec((_NC, 3, _BR), lambda i: (0, 0, i)),
            pl.BlockSpec((_BR, _D), lambda i: (i, 0)),
            pl.BlockSpec((_BR, _D), lambda i: (i, 0)),
            pl.BlockSpec((_BR, _D), lambda i: (i, 0)),
            pl.BlockSpec((_D, _H), lambda i: (0, 0)),
            pl.BlockSpec((_D, _H), lambda i: (0, 0)),
            pl.BlockSpec((_D, _H), lambda i: (0, 0)),
        ],
        out_specs=[
            pl.BlockSpec((3, _BR), lambda i: (0, i)),
            pl.BlockSpec((_BR, _H), lambda i: (i, 0)),
            pl.BlockSpec((_BR, _H), lambda i: (i, 0)),
            pl.BlockSpec((_BR, _H), lambda i: (i, 0)),
        ],
        out_shape=[
            jax.ShapeDtypeStruct((3, _NP), f32),
            jax.ShapeDtypeStruct((_N, _H), f32),
            jax.ShapeDtypeStruct((_N, _H), f32),
            jax.ShapeDtypeStruct((_N, _H), f32),
        ],
    )(degp, x1, x2, x3, W1, W2, W3)


def _tc_att_body(accD, xwd1, xwd2, xwd3, dinv, b1, b2, b3, Wf, bf,
                 W11, b11, W22, b22, W33, b33,
                 co1, co2, co3, zd1, zd2, zd3, base):
    embs, coefs, dvs = [], [], []
    for k, (xwd, b) in enumerate(((xwd1, b1), (xwd2, b2), (xwd3, b3))):
        dv = dinv[k][:, None]
        dvs.append(dv)
        pre = (accD[0, k] + accD[1, k] + xwd[...]) * dv + b[...][None, :]
        emb = jnp.maximum(pre, 0.0)
        embs.append(emb)
        s = jnp.sum(emb * Wf[...][:, 0][None, :], axis=1, keepdims=True)
        s = s + bf[...][None, :]
        coefs.append(jnp.exp(jnp.where(s >= 0, s, 0.01 * s)))
    den = coefs[0] + coefs[1] + coefs[2]
    for c_out, c in zip((co1, co2, co3), coefs):
        c_out[...] = c / den
    comb = (coefs[0] * embs[0] + coefs[1] * embs[1] + coefs[2] * embs[2]) / den
    acc_base = (b11[...] + b22[...] + b33[...])[None, :]
    for k, (W, zd_out) in enumerate(((W11, zd1), (W22, zd2), (W33, zd3))):
        z = jnp.dot(comb, W[...], preferred_element_type=jnp.float32)
        zd = z * dvs[k]
        zd_out[...] = zd
        acc_base = acc_base + zd * dvs[k]
    base[...] = acc_base


def _tc_att(accD, xwd1, xwd2, xwd3, dinv, b1, b2, b3, Wf, bf,
            W11, b11, W22, b22, W33, b33):
    f32 = jnp.float32
    vec = lambda n: pl.BlockSpec((n,), lambda i: (0,))
    return pl.pallas_call(
        _tc_att_body,
        grid=(_GRID,),
        in_specs=[
            pl.BlockSpec((_NC, 3, _BR, _H), lambda i: (0, 0, i, 0)),
            pl.BlockSpec((_BR, _H), lambda i: (i, 0)),
            pl.BlockSpec((_BR, _H), lambda i: (i, 0)),
            pl.BlockSpec((_BR, _H), lambda i: (i, 0)),
            pl.BlockSpec((3, _BR), lambda i: (0, i)),
            vec(_H), vec(_H), vec(_H),
            pl.BlockSpec((_H, 1), lambda i: (0, 0)),
            vec(1),
            pl.BlockSpec((_H, _O), lambda i: (0, 0)), vec(_O),
            pl.BlockSpec((_H, _O), lambda i: (0, 0)), vec(_O),
            pl.BlockSpec((_H, _O), lambda i: (0, 0)), vec(_O),
        ],
        out_specs=[
            pl.BlockSpec((_BR, 1), lambda i: (i, 0)),
            pl.BlockSpec((_BR, 1), lambda i: (i, 0)),
            pl.BlockSpec((_BR, 1), lambda i: (i, 0)),
            pl.BlockSpec((_BR, _O), lambda i: (i, 0)),
            pl.BlockSpec((_BR, _O), lambda i: (i, 0)),
            pl.BlockSpec((_BR, _O), lambda i: (i, 0)),
            pl.BlockSpec((_BR, _O), lambda i: (i, 0)),
        ],
        out_shape=[
            jax.ShapeDtypeStruct((_N, 1), f32),
            jax.ShapeDtypeStruct((_N, 1), f32),
            jax.ShapeDtypeStruct((_N, 1), f32),
            jax.ShapeDtypeStruct((_N, _O), f32),
            jax.ShapeDtypeStruct((_N, _O), f32),
            jax.ShapeDtypeStruct((_N, _O), f32),
            jax.ShapeDtypeStruct((_N, _O), f32),
        ],
    )(accD, xwd1, xwd2, xwd3, dinv, b1, b2, b3, Wf, bf,
      W11, b11, W22, b22, W33, b33)


def _tc_final_body(accF, dinv, base, out):
    acc = base[...]
    for k in range(3):
        acc = acc + (accF[0, k] + accF[1, k]) * dinv[k][:, None]
    out[...] = acc


def _tc_final(accF, dinv, base):
    return pl.pallas_call(
        _tc_final_body,
        grid=(_GRID,),
        in_specs=[
            pl.BlockSpec((_NC, 3, _BR, _O), lambda i: (0, 0, i, 0)),
            pl.BlockSpec((3, _BR), lambda i: (0, i)),
            pl.BlockSpec((_BR, _O), lambda i: (i, 0)),
        ],
        out_specs=pl.BlockSpec((_BR, _O), lambda i: (i, 0)),
        out_shape=jax.ShapeDtypeStruct((_N, _O), jnp.float32),
    )(accF, dinv, base)


def kernel(x1, edge_index1, edge_attr1, x2, edge_index2, edge_attr2,
           x3, edge_index3, edge_attr3,
           W1, b1, W2, b2, W3, b3, Wf, bf,
           W11, b11, W22, b22, W33, b33):
    f32 = jnp.float32
    pad = _EPAD - _E
    pad_idx = (jnp.arange(pad, dtype=jnp.int32) * 37) % _N
    rs = lambda a: jnp.concatenate([a, pad_idx]).reshape(_NW, _NCHUNK, _CH)
    rw = lambda a: jnp.concatenate(
        [a, jnp.zeros((pad,), a.dtype)]).reshape(_NW, _NCHUNK, _CH)
    r1, c1 = rs(edge_index1[0]), rs(edge_index1[1])
    r2, c2 = rs(edge_index2[0]), rs(edge_index2[1])
    r3, c3 = rs(edge_index3[0]), rs(edge_index3[1])
    e1, e2, e3 = rw(edge_attr1), rw(edge_attr2), rw(edge_attr3)

    zeros_np = jnp.zeros((_NP,), f32)
    zeros_h = jnp.zeros((_NP, _H), f32)
    zeros_o = jnp.zeros((_NP, _O), f32)

    degp = _sc_degrees(c1, c2, c3, e1, e2, e3, zeros_np).reshape(_NC, 3, _NP)
    dinv, xwd1, xwd2, xwd3 = _tc_prep(degp, x1, x2, x3, W1, W2, W3)
    accD = _sc_msg32(xwd1, xwd2, xwd3, r1, r2, r3, c1, c2, c3,
                     e1, e2, e3, zeros_h).reshape(_NC, 3, _NP, _H)
    co1, co2, co3, zd1, zd2, zd3, base = _tc_att(
        accD, xwd1, xwd2, xwd3, dinv, b1, b2, b3, Wf, bf,
        W11, b11, W22, b22, W33, b33)
    accF = _sc_msg16(zd1, zd2, zd3, r1, r2, r3, c1, c2, c3,
                     e1, e2, e3, zeros_o).reshape(_NC, 3, _NP, _O)
    out4 = _tc_final(accF, dinv, base)
    return (out4, co1, co2, co3)
